# Initial kernel scaffold; baseline (speedup 1.0000x reference)
import functools

import jax
import jax.numpy as jnp
from jax import lax
from jax.experimental import pallas as pl
from jax.experimental.pallas import tpu as pltpu
from jax.experimental.pallas import tpu_sc as plsc

N = 10000
D = 128
E = 320000

NC = 2
NS = 16
NW = NC * NS

CHUNK = 128
NCHUNK = 79
EPW = CHUNK * NCHUNK
EP = EPW * NW
NP = 10240
SHARD = NP // NS

_mesh = plsc.VectorSubcoreMesh(core_axis_name="c", subcore_axis_name="s")


def _msg_body(xs_hbm, srcr, dstr, zeros_hbm, out_hbm,
              src_v, dst_v, rows_v, shared_out, gsem):
    c = lax.axis_index("c")
    s = lax.axis_index("s")
    wid = s * NC + c
    pltpu.sync_copy(zeros_hbm, shared_out.at[pl.ds(s * SHARD, SHARD)])
    pltpu.sync_copy(srcr.at[wid], src_v)
    pltpu.sync_copy(dstr.at[wid], dst_v)
    plsc.subcore_barrier()

    def body(j, carry):
        pltpu.async_copy(xs_hbm.at[src_v.at[j]], rows_v, gsem).wait()
        pltpu.sync_copy(rows_v, shared_out.at[dst_v.at[j]], add=True)
        return carry

    lax.fori_loop(0, NCHUNK, body, 0, unroll=False)
    plsc.subcore_barrier()
    pltpu.sync_copy(shared_out.at[pl.ds(s * SHARD, SHARD)],
                    out_hbm.at[c, pl.ds(s * SHARD, SHARD)])


def _make_msg_kernel(width):
    return pl.kernel(
        _msg_body,
        out_type=jax.ShapeDtypeStruct((NC, NP, width), jnp.float32),
        mesh=_mesh,
        scratch_types=[
            pltpu.VMEM((NCHUNK, CHUNK), jnp.int32),
            pltpu.VMEM((NCHUNK, CHUNK), jnp.int32),
            pltpu.VMEM((CHUNK, width), jnp.float32),
            pltpu.VMEM_SHARED((NP, width), jnp.float32),
            pltpu.SemaphoreType.DMA,
        ],
    )


_msg_pass = _make_msg_kernel(D)
_deg_pass = _make_msg_kernel(16)


def _dv(degp):
    return lax.rsqrt(1.0 + degp[0][:, 0:1] + degp[1][:, 0:1])


def _tc_first_body(x_ref, w_ref, deg_ref, xs_ref):
    dv = _dv(deg_ref)
    xs_ref[...] = dv * jnp.dot(x_ref[...], w_ref[...],
                               preferred_element_type=jnp.float32)


def _tc_mid_body(p_ref, xs_ref, deg_ref, b_ref, w_ref, o_ref):
    dv = _dv(deg_ref)
    h = dv * (p_ref[0] + p_ref[1] + xs_ref[...]) + b_ref[...]
    h = jnp.maximum(h, 0.0)
    o_ref[...] = dv * jnp.dot(h, w_ref[...],
                              preferred_element_type=jnp.float32)


def _tc_last_body(p_ref, xs_ref, deg_ref, b_ref, o_ref):
    dv = _dv(deg_ref)
    o_ref[...] = dv * (p_ref[0] + p_ref[1] + xs_ref[...]) + b_ref[...]


BLK = 512
GRID = NP // BLK

_row_spec = pl.BlockSpec((BLK, D), lambda i: (i, 0))
_p_spec = pl.BlockSpec((NC, BLK, D), lambda i: (0, i, 0))
_deg_spec = pl.BlockSpec((NC, BLK, 16), lambda i: (0, i, 0))
_w_spec = pl.BlockSpec((D, D), lambda i: (0, 0))
_b_spec = pl.BlockSpec((1, D), lambda i: (0, 0))
_out_struct = jax.ShapeDtypeStruct((NP, D), jnp.float32)

_tc_first = pl.pallas_call(
    _tc_first_body,
    grid=(GRID,),
    in_specs=[_row_spec, _w_spec, _deg_spec],
    out_specs=_row_spec,
    out_shape=_out_struct,
)

_tc_mid = pl.pallas_call(
    _tc_mid_body,
    grid=(GRID,),
    in_specs=[_p_spec, _row_spec, _deg_spec, _b_spec, _w_spec],
    out_specs=_row_spec,
    out_shape=_out_struct,
)

_tc_last = pl.pallas_call(
    _tc_last_body,
    grid=(GRID,),
    in_specs=[_p_spec, _row_spec, _deg_spec, _b_spec],
    out_specs=_row_spec,
    out_shape=_out_struct,
)


@jax.jit
def _run(x, edge_index, W1, b1, W2, b2, W3, b3):
    pad = EP - E
    src = jnp.concatenate([edge_index[0], jnp.full((pad,), N, jnp.int32)])
    dst = jnp.concatenate([edge_index[1], jnp.full((pad,), N, jnp.int32)])
    srcr = src.reshape(NW, NCHUNK, CHUNK)
    dstr = dst.reshape(NW, NCHUNK, CHUNK)

    xp = jnp.zeros((NP, D), jnp.float32).at[:N].set(x)
    zeros128 = jnp.zeros((SHARD, D), jnp.float32)
    zeros16 = jnp.zeros((SHARD, 16), jnp.float32)
    ones16 = jnp.ones((CHUNK, 16), jnp.float32)

    degp = _deg_pass(ones16, srcr, srcr, zeros16)

    xs = _tc_first(xp, W1, degp)
    p = _msg_pass(xs, srcr, dstr, zeros128)
    xs = _tc_mid(p, xs, degp, b1.reshape(1, D), W2)
    p = _msg_pass(xs, srcr, dstr, zeros128)
    xs = _tc_mid(p, xs, degp, b2.reshape(1, D), W3)
    p = _msg_pass(xs, srcr, dstr, zeros128)
    out = _tc_last(p, xs, degp, b3.reshape(1, D))
    return out[:N]


def kernel(x, edge_index, cache_name, W1, b1, W2, b2, W3, b3):
    return _run(x, edge_index, W1, b1, W2, b2, W3, b3)


# speedup vs baseline: 10.9201x; 10.9201x over previous
import functools

import jax
import jax.numpy as jnp
from jax import lax
from jax.experimental import pallas as pl
from jax.experimental.pallas import tpu as pltpu
from jax.experimental.pallas import tpu_sc as plsc

N = 10000
D = 128
E = 320000

NC = 2
NS = 16
NW = NC * NS

CHUNK = 128
NCHUNK = 79
EPW = CHUNK * NCHUNK
EP = EPW * NW
NP = 10240
SHARD = NP // NS

_mesh = plsc.VectorSubcoreMesh(core_axis_name="c", subcore_axis_name="s")


def _msg_body(xs_hbm, srcr, dstr, zeros_hbm, out_hbm,
              src_v, dst_v, rows_v, shared_out, gsem):
    c = lax.axis_index("c")
    s = lax.axis_index("s")
    wid = s * NC + c
    pltpu.sync_copy(zeros_hbm, shared_out.at[pl.ds(s * SHARD, SHARD)])
    pltpu.sync_copy(srcr.at[wid], src_v)
    pltpu.sync_copy(dstr.at[wid], dst_v)
    plsc.subcore_barrier()

    def body(j, carry):
        pltpu.async_copy(xs_hbm.at[src_v.at[j]], rows_v, gsem).wait()
        pltpu.sync_copy(rows_v, shared_out.at[dst_v.at[j]], add=True)
        return carry

    lax.fori_loop(0, NCHUNK, body, 0, unroll=False)
    plsc.subcore_barrier()
    pltpu.sync_copy(shared_out.at[pl.ds(s * SHARD, SHARD)],
                    out_hbm.at[c, pl.ds(s * SHARD, SHARD)])


def _deg_body(ones_hbm, srcr, zeros_hbm, out_hbm,
              src_v, rows_v, shared_out):
    c = lax.axis_index("c")
    s = lax.axis_index("s")
    wid = s * NC + c
    pltpu.sync_copy(zeros_hbm, shared_out.at[pl.ds(s * SHARD, SHARD)])
    pltpu.sync_copy(srcr.at[wid], src_v)
    pltpu.sync_copy(ones_hbm, rows_v)
    plsc.subcore_barrier()

    def body(j, carry):
        pltpu.sync_copy(rows_v, shared_out.at[src_v.at[j]], add=True)
        return carry

    lax.fori_loop(0, NCHUNK, body, 0, unroll=False)
    plsc.subcore_barrier()
    pltpu.sync_copy(shared_out.at[pl.ds(s * SHARD, SHARD)],
                    out_hbm.at[c, pl.ds(s * SHARD, SHARD)])


_msg_pass = pl.kernel(
    _msg_body,
    out_type=jax.ShapeDtypeStruct((NC, NP, D), jnp.float32),
    mesh=_mesh,
    scratch_types=[
        pltpu.VMEM((NCHUNK, CHUNK), jnp.int32),
        pltpu.VMEM((NCHUNK, CHUNK), jnp.int32),
        pltpu.VMEM((CHUNK, D), jnp.float32),
        pltpu.VMEM_SHARED((NP, D), jnp.float32),
        pltpu.SemaphoreType.DMA,
    ],
)

_deg_pass = pl.kernel(
    _deg_body,
    out_type=jax.ShapeDtypeStruct((NC, NP, D), jnp.float32),
    mesh=_mesh,
    scratch_types=[
        pltpu.VMEM((NCHUNK, CHUNK), jnp.int32),
        pltpu.VMEM((CHUNK, D), jnp.float32),
        pltpu.VMEM_SHARED((NP, D), jnp.float32),
    ],
)


def _dv(degp):
    return lax.rsqrt(1.0 + degp[0][:, 0:1] + degp[1][:, 0:1])


def _tc_first_body(x_ref, w_ref, deg_ref, xs_ref):
    dv = _dv(deg_ref)
    xs_ref[...] = dv * jnp.dot(x_ref[...], w_ref[...],
                               preferred_element_type=jnp.float32)


def _tc_mid_body(p_ref, xs_ref, deg_ref, b_ref, w_ref, o_ref):
    dv = _dv(deg_ref)
    h = dv * (p_ref[0] + p_ref[1] + xs_ref[...]) + b_ref[...]
    h = jnp.maximum(h, 0.0)
    o_ref[...] = dv * jnp.dot(h, w_ref[...],
                              preferred_element_type=jnp.float32)


def _tc_last_body(p_ref, xs_ref, deg_ref, b_ref, o_ref):
    dv = _dv(deg_ref)
    o_ref[...] = dv * (p_ref[0] + p_ref[1] + xs_ref[...]) + b_ref[...]


BLK = 512
GRID = NP // BLK

_row_spec = pl.BlockSpec((BLK, D), lambda i: (i, 0))
_p_spec = pl.BlockSpec((NC, BLK, D), lambda i: (0, i, 0))
_deg_spec = pl.BlockSpec((NC, BLK, D), lambda i: (0, i, 0))
_w_spec = pl.BlockSpec((D, D), lambda i: (0, 0))
_b_spec = pl.BlockSpec((1, D), lambda i: (0, 0))
_out_struct = jax.ShapeDtypeStruct((NP, D), jnp.float32)

_tc_first = pl.pallas_call(
    _tc_first_body,
    grid=(GRID,),
    in_specs=[_row_spec, _w_spec, _deg_spec],
    out_specs=_row_spec,
    out_shape=_out_struct,
)

_tc_mid = pl.pallas_call(
    _tc_mid_body,
    grid=(GRID,),
    in_specs=[_p_spec, _row_spec, _deg_spec, _b_spec, _w_spec],
    out_specs=_row_spec,
    out_shape=_out_struct,
)

_tc_last = pl.pallas_call(
    _tc_last_body,
    grid=(GRID,),
    in_specs=[_p_spec, _row_spec, _deg_spec, _b_spec],
    out_specs=_row_spec,
    out_shape=_out_struct,
)


@jax.jit
def _run(x, edge_index, W1, b1, W2, b2, W3, b3):
    pad = EP - E
    src = jnp.concatenate([edge_index[0], jnp.full((pad,), N, jnp.int32)])
    dst = jnp.concatenate([edge_index[1], jnp.full((pad,), N, jnp.int32)])
    srcr = src.reshape(NW, NCHUNK, CHUNK)
    dstr = dst.reshape(NW, NCHUNK, CHUNK)

    xp = jnp.zeros((NP, D), jnp.float32).at[:N].set(x)
    zeros128 = jnp.zeros((SHARD, D), jnp.float32)
    ones128 = jnp.ones((CHUNK, D), jnp.float32)

    degp = _deg_pass(ones128, srcr, zeros128)

    xs = _tc_first(xp, W1, degp)
    p = _msg_pass(xs, srcr, dstr, zeros128)
    xs = _tc_mid(p, xs, degp, b1.reshape(1, D), W2)
    p = _msg_pass(xs, srcr, dstr, zeros128)
    xs = _tc_mid(p, xs, degp, b2.reshape(1, D), W3)
    p = _msg_pass(xs, srcr, dstr, zeros128)
    out = _tc_last(p, xs, degp, b3.reshape(1, D))
    return out[:N]


def kernel(x, edge_index, cache_name, W1, b1, W2, b2, W3, b3):
    return _run(x, edge_index, W1, b1, W2, b2, W3, b3)
